# baseline (speedup 1.0000x reference)
import functools

import numpy as np
import jax
import jax.numpy as jnp
from jax import lax
from jax.experimental import pallas as pl
from jax.experimental.pallas import tpu as pltpu
from jax.experimental.pallas import tpu_sc as plsc

B = 2
DIM = 384
HEADS = 8
HD = DIM // HEADS
TOPK = 8
H = 48
W = 48
HU = 24
WU = 24
N = H * W
NU = HU * WU
SCALE = HD ** -0.5
F32 = jnp.float32


def _prep_body(x_ref, u_ref, wq_ref, bq_ref, wkv_ref, bkv_ref, pt_ref,
               gum_ref, bm_ref, q_ref, kvu_ref, gs_ref):
    x = x_ref[0]
    u = u_ref[0]
    xq = jnp.dot(x, wq_ref[...], preferred_element_type=F32) + bq_ref[...]
    kvu = jnp.dot(u, wkv_ref[...], preferred_element_type=F32) + bkv_ref[...]
    for h in range(HEADS):
        q_ref[0, h] = xq[:, h * HD:(h + 1) * HD]
        kvu_ref[0, h] = kvu[:, h * 2 * HD:(h + 1) * 2 * HD]
    k_all = jnp.concatenate(
        [kvu[:, h * 2 * HD:h * 2 * HD + HD] for h in range(HEADS)], axis=1)
    qsum = jnp.sum(xq, axis=0, keepdims=True)
    xmean = jnp.broadcast_to(
        jnp.mean(x, axis=1, keepdims=True), (N, 128))
    heat = jnp.dot(pt_ref[...], xmean,
                   preferred_element_type=F32)[:, :1]
    gs0 = jnp.dot(k_all * qsum * heat, bm_ref[...],
                  preferred_element_type=F32)[:, :HEADS]
    gs_ref[0] = gs0 * (SCALE / N) + gum_ref[0]


def _prep_call(x_tok, u_tok, wq, bq, wkv, bkv, pt, gum, bm):
    return pl.pallas_call(
        _prep_body,
        grid=(B,),
        in_specs=[
            pl.BlockSpec((1, N, DIM), lambda b: (b, 0, 0)),
            pl.BlockSpec((1, NU, DIM), lambda b: (b, 0, 0)),
            pl.BlockSpec((DIM, DIM), lambda b: (0, 0)),
            pl.BlockSpec((1, DIM), lambda b: (0, 0)),
            pl.BlockSpec((DIM, 2 * DIM), lambda b: (0, 0)),
            pl.BlockSpec((1, 2 * DIM), lambda b: (0, 0)),
            pl.BlockSpec((NU, N), lambda b: (0, 0)),
            pl.BlockSpec((1, NU, HEADS), lambda b: (b, 0, 0)),
            pl.BlockSpec((DIM, 128), lambda b: (0, 0)),
        ],
        out_specs=[
            pl.BlockSpec((1, HEADS, N, HD), lambda b: (b, 0, 0, 0)),
            pl.BlockSpec((1, HEADS, NU, 2 * HD), lambda b: (b, 0, 0, 0)),
            pl.BlockSpec((1, NU, HEADS), lambda b: (b, 0, 0)),
        ],
        out_shape=[
            jax.ShapeDtypeStruct((B, HEADS, N, HD), F32),
            jax.ShapeDtypeStruct((B, HEADS, NU, 2 * HD), F32),
            jax.ShapeDtypeStruct((B, NU, HEADS), F32),
        ],
    )(x_tok, u_tok, wq, bq, wkv, bkv, pt, gum, bm)


_NEG = -1.0e30


def _sc_topk_gather(gs_hbm, x_hbm, out_hbm, gs_v, idx_v, rows_v, sem):
    cid = lax.axis_index("c")
    sid = lax.axis_index("s")
    wid = sid * 2 + cid

    @pl.when(wid < B * HEADS)
    def _():
        bh = wid
        pltpu.sync_copy(gs_hbm.at[bh], gs_v)
        lanes = lax.iota(jnp.int32, 16)
        lt8 = lanes < 8

        def body(i, carry):
            cur_v, cur_i = carry
            cv = gs_v[pl.ds(i * 16, 16)]
            ci = lanes + i * 16
            nd_v, nd_i = plsc.sort_key_val(cv, ci, descending=True)
            ca_v, ca_i = plsc.sort_key_val(cur_v, cur_i, descending=False)
            comb_v = jnp.where(lt8, nd_v, ca_v)
            comb_i = jnp.where(lt8, nd_i, ca_i)
            m_v, m_i = plsc.sort_key_val(comb_v, comb_i, descending=True)
            return jnp.where(lt8, m_v, _NEG), m_i

        cur_v = jnp.full((16,), _NEG, F32)
        cur_i = jnp.zeros((16,), jnp.int32)
        _, top_i = lax.fori_loop(0, NU // 16, body, (cur_v, cur_i))

        hu = top_i // WU
        wu = top_i % WU
        base = hu * (2 * W) + wu * 2 + (bh // HEADS) * N
        for j, off in enumerate((0, 1, W, W + 1)):
            plsc.store_scatter(idx_v, [lanes + j * 8], base + off, mask=lt8)
        pltpu.async_copy(x_hbm.at[idx_v], rows_v, sem).wait()
        pltpu.sync_copy(rows_v, out_hbm.at[pl.ds(bh * 4 * TOPK, 4 * TOPK)])


@functools.cache
def _sc_call_fn():
    mesh = plsc.VectorSubcoreMesh(core_axis_name="c", subcore_axis_name="s")
    return pl.kernel(
        _sc_topk_gather,
        out_type=jax.ShapeDtypeStruct((B * HEADS * 4 * TOPK, DIM), F32),
        mesh=mesh,
        compiler_params=pltpu.CompilerParams(needs_layout_passes=False),
        scratch_types=[
            pltpu.VMEM((NU,), F32),
            pltpu.VMEM((4 * TOPK,), jnp.int32),
            pltpu.VMEM((4 * TOPK, DIM), F32),
            pltpu.SemaphoreType.DMA,
        ],
    )


def _sc_call(gs16, x_flat):
    return _sc_call_fn()(gs16, x_flat)


def _attn_body(q_ref, kvu_ref, gx_ref, wk_ref, bk_ref, gw_ref, gb_ref,
               rb_ref, wp_ref, bp_ref, pe_ref, peb_ref, o_ref):
    v_all = jnp.concatenate(
        [kvu_ref[0, h, :, HD:] for h in range(HEADS)], axis=1)
    z = jnp.zeros((80, DIM), F32)
    vpad = jnp.concatenate([z, v_all, z], axis=0)
    wu = lax.broadcasted_iota(jnp.int32, (NU, 1), 0) % WU
    acc = jnp.zeros((NU, DIM), F32)
    for dy in range(7):
        for dx in range(7):
            off = (dy - 3) * WU + (dx - 3)
            sh = vpad[80 + off:80 + off + NU]
            tap = pe_ref[dy * 7 + dx:dy * 7 + dx + 1, :]
            o = dx - 3
            if o != 0:
                valid = jnp.logical_and(wu + o >= 0, wu + o <= WU - 1)
                sh = sh * jnp.where(valid, 1.0, 0.0)
            acc = acc + sh * tap
    vpe = acc + peb_ref[...]
    ps = jnp.dot(vpe, wp_ref[...], preferred_element_type=F32)
    out_acc = lax.dot_general(rb_ref[...], ps.astype(jnp.bfloat16),
                              (((1,), (0,)), ((), ())),
                              preferred_element_type=F32)
    out_acc = out_acc + bp_ref[...]

    for h in range(HEADS):
        qh = q_ref[0, h]
        kv = kvu_ref[0, h]
        k = kv[:, :HD]
        v = kv[:, HD:]
        sim = lax.dot_general(qh, k, (((1,), (1,)), ((), ())),
                              preferred_element_type=F32) * SCALE
        m = jnp.max(sim, axis=1, keepdims=True)
        e = jnp.exp(sim - m)
        s = jnp.sum(e, axis=1, keepdims=True)
        coarse = lax.dot_general(e, v, (((1,), (0,)), ((), ())),
                                 preferred_element_type=F32) / s

        gx = gx_ref[0, h]
        tkv = jnp.dot(gx, wk_ref[h], preferred_element_type=F32) + bk_ref[h]
        tk = tkv[:, :HD]
        tv = tkv[:, HD:]
        fsim = lax.dot_general(qh, tk, (((1,), (1,)), ((), ())),
                               preferred_element_type=F32) * SCALE
        fm = jnp.max(fsim, axis=1, keepdims=True)
        fe = jnp.exp(fsim - fm)
        fs = jnp.sum(fe, axis=1, keepdims=True)
        refined = lax.dot_general(fe, tv, (((1,), (0,)), ((), ())),
                                  preferred_element_type=F32) / fs

        cc = jnp.concatenate([coarse, refined], axis=1)
        gate = jax.nn.sigmoid(
            jnp.dot(cc, gw_ref[...], preferred_element_type=F32)
            + gb_ref[...])
        out_h = gate * refined + (1.0 - gate) * coarse
        out_acc = out_acc + jnp.dot(
            out_h, wp_ref[h * HD:(h + 1) * HD, :],
            preferred_element_type=F32)

    o_ref[0] = out_acc


def _attn_call(q4, kvu4, gx4, wkvh, bkvh, gwT, gb, rbig, wp, bp, peT, peB):
    return pl.pallas_call(
        _attn_body,
        grid=(B,),
        in_specs=[
            pl.BlockSpec((1, HEADS, N, HD), lambda b: (b, 0, 0, 0)),
            pl.BlockSpec((1, HEADS, NU, 2 * HD), lambda b: (b, 0, 0, 0)),
            pl.BlockSpec((1, HEADS, 4 * TOPK, DIM), lambda b: (b, 0, 0, 0)),
            pl.BlockSpec((HEADS, DIM, 2 * HD), lambda b: (0, 0, 0)),
            pl.BlockSpec((HEADS, 1, 2 * HD), lambda b: (0, 0, 0)),
            pl.BlockSpec((2 * HD, HD), lambda b: (0, 0)),
            pl.BlockSpec((1, HD), lambda b: (0, 0)),
            pl.BlockSpec((N, NU), lambda b: (0, 0)),
            pl.BlockSpec((DIM, DIM), lambda b: (0, 0)),
            pl.BlockSpec((1, DIM), lambda b: (0, 0)),
            pl.BlockSpec((49, DIM), lambda b: (0, 0)),
            pl.BlockSpec((1, DIM), lambda b: (0, 0)),
        ],
        out_specs=pl.BlockSpec((1, N, DIM), lambda b: (b, 0, 0)),
        out_shape=jax.ShapeDtypeStruct((B, N, DIM), F32),
    )(q4, kvu4, gx4, wkvh, bkvh, gwT, gb, rbig, wp, bp, peT, peB)


def _pool_matrix():
    n = np.arange(N)
    m = (n // (2 * W)) * WU + (n % W) // 2
    pt = np.zeros((NU, N), np.float32)
    pt[m, n] = 0.25
    return jnp.asarray(pt)


def _block_mask():
    bm = np.zeros((DIM, 128), np.float32)
    bm[np.arange(DIM), np.arange(DIM) // HD] = 1.0
    return jnp.asarray(bm)


def _resize_kron():
    r = jax.image.resize(jnp.eye(HU, dtype=F32), (H, HU), method="bilinear")
    return jnp.kron(r, r).astype(jnp.bfloat16)


def kernel(x, upper_feat, q_w, q_g, q_b, kv_w, kv_g, kv_b,
           proj_w, proj_g, proj_b, pe_w, pe_g, pe_b, gate_w, gate_b):
    wq = (q_w[:, :, 0, 0] * q_g[:, None]).T
    bq = q_b[None, :]
    wkv = (kv_w[:, :, 0, 0] * kv_g[:, None]).T
    bkv = kv_b[None, :]
    wp = (proj_w[:, :, 0, 0] * proj_g[:, None]).T
    bp = proj_b[None, :]
    peT = pe_w[:, 0].reshape(DIM, 49).T * pe_g[None, :]
    peB = pe_b[None, :]
    gwT = gate_w.T
    gb = gate_b[None, :]
    wkvh = wkv.reshape(DIM, HEADS, 2 * HD).transpose(1, 0, 2)
    bkvh = kv_b.reshape(HEADS, 1, 2 * HD)

    gum = jax.random.gumbel(
        jax.random.key(42), (B, HEADS, NU), F32).transpose(0, 2, 1)

    x_tok = x.reshape(B, DIM, N).transpose(0, 2, 1)
    u_tok = upper_feat.reshape(B, DIM, NU).transpose(0, 2, 1)

    q4, kvu4, gs = _prep_call(x_tok, u_tok, wq, bq, wkv, bkv,
                              _pool_matrix(), gum, _block_mask())
    gs16 = gs.transpose(0, 2, 1).reshape(B * HEADS, NU)
    gx = _sc_call(gs16, x_tok.reshape(B * N, DIM))
    gx4 = gx.reshape(B, HEADS, 4 * TOPK, DIM)
    out_tok = _attn_call(q4, kvu4, gx4, wkvh, bkvh, gwT, gb,
                         _resize_kron(), wp, bp, peT, peB)
    return out_tok.transpose(0, 2, 1).reshape(B, DIM, H, W)


# speedup vs baseline: 1.2041x; 1.2041x over previous
import functools

import numpy as np
import jax
import jax.numpy as jnp
from jax import lax
from jax.experimental import pallas as pl
from jax.experimental.pallas import tpu as pltpu
from jax.experimental.pallas import tpu_sc as plsc

B = 2
DIM = 384
HEADS = 8
HD = DIM // HEADS
TOPK = 8
H = 48
W = 48
HU = 24
WU = 24
N = H * W
NU = HU * WU
SCALE = HD ** -0.5
F32 = jnp.float32


def _prep_body(x_ref, u_ref, wq_ref, bq_ref, wkv_ref, bkv_ref, pt_ref,
               gum_ref, bm_ref, q_ref, kvu_ref, gs_ref):
    x = x_ref[0]
    u = u_ref[0]
    xq = jnp.dot(x, wq_ref[...], preferred_element_type=F32) + bq_ref[...]
    kvu = jnp.dot(u, wkv_ref[...], preferred_element_type=F32) + bkv_ref[...]
    for h in range(HEADS):
        q_ref[0, h] = xq[:, h * HD:(h + 1) * HD]
        kvu_ref[0, h] = kvu[:, h * 2 * HD:(h + 1) * 2 * HD]
    k_all = jnp.concatenate(
        [kvu[:, h * 2 * HD:h * 2 * HD + HD] for h in range(HEADS)], axis=1)
    qsum = jnp.sum(xq, axis=0, keepdims=True)
    xmean = jnp.broadcast_to(
        jnp.mean(x, axis=1, keepdims=True), (N, 128))
    heat = jnp.dot(pt_ref[...], xmean,
                   preferred_element_type=F32)[:, :1]
    gs0 = jnp.dot(k_all * qsum * heat, bm_ref[...],
                  preferred_element_type=F32)[:, :HEADS]
    gs_ref[0] = gs0 * (SCALE / N) + gum_ref[0]


def _prep_call(x_tok, u_tok, wq, bq, wkv, bkv, pt, gum, bm):
    return pl.pallas_call(
        _prep_body,
        grid=(B,),
        in_specs=[
            pl.BlockSpec((1, N, DIM), lambda b: (b, 0, 0)),
            pl.BlockSpec((1, NU, DIM), lambda b: (b, 0, 0)),
            pl.BlockSpec((DIM, DIM), lambda b: (0, 0)),
            pl.BlockSpec((1, DIM), lambda b: (0, 0)),
            pl.BlockSpec((DIM, 2 * DIM), lambda b: (0, 0)),
            pl.BlockSpec((1, 2 * DIM), lambda b: (0, 0)),
            pl.BlockSpec((NU, N), lambda b: (0, 0)),
            pl.BlockSpec((1, NU, HEADS), lambda b: (b, 0, 0)),
            pl.BlockSpec((DIM, 128), lambda b: (0, 0)),
        ],
        out_specs=[
            pl.BlockSpec((1, HEADS, N, HD), lambda b: (b, 0, 0, 0)),
            pl.BlockSpec((1, HEADS, NU, 2 * HD), lambda b: (b, 0, 0, 0)),
            pl.BlockSpec((1, NU, HEADS), lambda b: (b, 0, 0)),
        ],
        out_shape=[
            jax.ShapeDtypeStruct((B, HEADS, N, HD), F32),
            jax.ShapeDtypeStruct((B, HEADS, NU, 2 * HD), F32),
            jax.ShapeDtypeStruct((B, NU, HEADS), F32),
        ],
    )(x_tok, u_tok, wq, bq, wkv, bkv, pt, gum, bm)


_NEG = -1.0e30


def _sc_topk_gather(gs_hbm, x_hbm, out_hbm, gs_v, idx_v, rows_v, sem):
    cid = lax.axis_index("c")
    sid = lax.axis_index("s")
    wid = sid * 2 + cid

    @pl.when(wid < B * HEADS)
    def _():
        bh = wid
        pltpu.sync_copy(gs_hbm.at[bh], gs_v)
        lanes = lax.iota(jnp.int32, 16)
        lt8 = lanes < 8

        def body(i, carry):
            cur_v, cur_i = carry
            cv = gs_v[pl.ds(i * 16, 16)]
            ci = lanes + i * 16
            nd_v, nd_i = plsc.sort_key_val(cv, ci, descending=True)
            ca_v, ca_i = plsc.sort_key_val(cur_v, cur_i, descending=False)
            comb_v = jnp.where(lt8, nd_v, ca_v)
            comb_i = jnp.where(lt8, nd_i, ca_i)
            m_v, m_i = plsc.sort_key_val(comb_v, comb_i, descending=True)
            return jnp.where(lt8, m_v, _NEG), m_i

        cur_v = jnp.full((16,), _NEG, F32)
        cur_i = jnp.zeros((16,), jnp.int32)
        _, top_i = lax.fori_loop(0, NU // 16, body, (cur_v, cur_i))

        hu = top_i // WU
        wu = top_i % WU
        base = hu * (2 * W) + wu * 2 + (bh // HEADS) * N
        for j, off in enumerate((0, 1, W, W + 1)):
            plsc.store_scatter(idx_v, [lanes + j * 8], base + off, mask=lt8)
        pltpu.async_copy(x_hbm.at[idx_v], rows_v, sem).wait()
        pltpu.sync_copy(rows_v, out_hbm.at[pl.ds(bh * 4 * TOPK, 4 * TOPK)])


@functools.cache
def _sc_call_fn():
    mesh = plsc.VectorSubcoreMesh(core_axis_name="c", subcore_axis_name="s")
    return pl.kernel(
        _sc_topk_gather,
        out_type=jax.ShapeDtypeStruct((B * HEADS * 4 * TOPK, DIM), F32),
        mesh=mesh,
        compiler_params=pltpu.CompilerParams(needs_layout_passes=False),
        scratch_types=[
            pltpu.VMEM((NU,), F32),
            pltpu.VMEM((4 * TOPK,), jnp.int32),
            pltpu.VMEM((4 * TOPK, DIM), F32),
            pltpu.SemaphoreType.DMA,
        ],
    )


def _sc_call(gs16, x_flat):
    return _sc_call_fn()(gs16, x_flat)


def _attn_body(q_ref, kvu_ref, gx_ref, wk_ref, bk_ref, gw_ref, gb_ref, o_ref):
    qh = q_ref[0, 0]
    kv = kvu_ref[0, 0]
    k = kv[:, :HD]
    v = kv[:, HD:]
    sim = lax.dot_general(qh, k, (((1,), (1,)), ((), ())),
                          preferred_element_type=F32) * SCALE
    m = jnp.max(sim, axis=1, keepdims=True)
    e = jnp.exp(sim - m)
    s = jnp.sum(e, axis=1, keepdims=True)
    coarse = lax.dot_general(e, v, (((1,), (0,)), ((), ())),
                             preferred_element_type=F32) / s

    gx = gx_ref[0, 0]
    tkv = jnp.dot(gx, wk_ref[0], preferred_element_type=F32) + bk_ref[0]
    tk = tkv[:, :HD]
    tv = tkv[:, HD:]
    fsim = lax.dot_general(qh, tk, (((1,), (1,)), ((), ())),
                           preferred_element_type=F32) * SCALE
    fm = jnp.max(fsim, axis=1, keepdims=True)
    fe = jnp.exp(fsim - fm)
    fs = jnp.sum(fe, axis=1, keepdims=True)
    refined = lax.dot_general(fe, tv, (((1,), (0,)), ((), ())),
                              preferred_element_type=F32) / fs

    cc = jnp.concatenate([coarse, refined], axis=1)
    gate = jax.nn.sigmoid(
        jnp.dot(cc, gw_ref[...], preferred_element_type=F32) + gb_ref[...])
    o_ref[0, 0] = gate * refined + (1.0 - gate) * coarse


def _attn_call(q4, kvu4, gx4, wkvh, bkvh, gwT, gb):
    return pl.pallas_call(
        _attn_body,
        grid=(B, HEADS),
        in_specs=[
            pl.BlockSpec((1, 1, N, HD), lambda b, h: (b, h, 0, 0)),
            pl.BlockSpec((1, 1, NU, 2 * HD), lambda b, h: (b, h, 0, 0)),
            pl.BlockSpec((1, 1, 4 * TOPK, DIM), lambda b, h: (b, h, 0, 0)),
            pl.BlockSpec((1, DIM, 2 * HD), lambda b, h: (h, 0, 0)),
            pl.BlockSpec((1, 1, 2 * HD), lambda b, h: (h, 0, 0)),
            pl.BlockSpec((2 * HD, HD), lambda b, h: (0, 0)),
            pl.BlockSpec((1, HD), lambda b, h: (0, 0)),
        ],
        out_specs=pl.BlockSpec((1, 1, N, HD), lambda b, h: (b, h, 0, 0)),
        out_shape=jax.ShapeDtypeStruct((B, HEADS, N, HD), F32),
    )(q4, kvu4, gx4, wkvh, bkvh, gwT, gb)


def _epi_body(oa_ref, kvu_ref, rb_ref, wp_ref, bp_ref, pe_ref, peb_ref, o_ref):
    v_all = jnp.concatenate(
        [kvu_ref[0, h, :, HD:] for h in range(HEADS)], axis=1)
    z = jnp.zeros((80, DIM), F32)
    vpad = jnp.concatenate([z, v_all, z], axis=0)
    wu = lax.broadcasted_iota(jnp.int32, (NU, 1), 0) % WU
    acc = jnp.zeros((NU, DIM), F32)
    for dy in range(7):
        for dx in range(7):
            off = (dy - 3) * WU + (dx - 3)
            sh = vpad[80 + off:80 + off + NU]
            tap = pe_ref[dy * 7 + dx:dy * 7 + dx + 1, :]
            o = dx - 3
            if o != 0:
                valid = jnp.logical_and(wu + o >= 0, wu + o <= WU - 1)
                sh = sh * jnp.where(valid, 1.0, 0.0)
            acc = acc + sh * tap
    vpe = acc + peb_ref[...]
    ps = jnp.dot(vpe, wp_ref[...], preferred_element_type=F32)
    rb = lax.dot_general(rb_ref[...], ps.astype(jnp.bfloat16),
                         (((1,), (0,)), ((), ())),
                         preferred_element_type=F32)
    attn_tok = jnp.concatenate(
        [oa_ref[0, h] for h in range(HEADS)], axis=1)
    o_ref[0] = (jnp.dot(attn_tok, wp_ref[...], preferred_element_type=F32)
                + rb + bp_ref[...])


def _epi_call(oa, kvu4, rbig, wp, bp, peT, peB):
    return pl.pallas_call(
        _epi_body,
        grid=(B,),
        in_specs=[
            pl.BlockSpec((1, HEADS, N, HD), lambda b: (b, 0, 0, 0)),
            pl.BlockSpec((1, HEADS, NU, 2 * HD), lambda b: (b, 0, 0, 0)),
            pl.BlockSpec((N, NU), lambda b: (0, 0)),
            pl.BlockSpec((DIM, DIM), lambda b: (0, 0)),
            pl.BlockSpec((1, DIM), lambda b: (0, 0)),
            pl.BlockSpec((49, DIM), lambda b: (0, 0)),
            pl.BlockSpec((1, DIM), lambda b: (0, 0)),
        ],
        out_specs=pl.BlockSpec((1, N, DIM), lambda b: (b, 0, 0)),
        out_shape=jax.ShapeDtypeStruct((B, N, DIM), F32),
    )(oa, kvu4, rbig, wp, bp, peT, peB)


def _pool_matrix():
    n = np.arange(N)
    m = (n // (2 * W)) * WU + (n % W) // 2
    pt = np.zeros((NU, N), np.float32)
    pt[m, n] = 0.25
    return jnp.asarray(pt)


def _block_mask():
    bm = np.zeros((DIM, 128), np.float32)
    bm[np.arange(DIM), np.arange(DIM) // HD] = 1.0
    return jnp.asarray(bm)


def _resize_kron():
    r = jax.image.resize(jnp.eye(HU, dtype=F32), (H, HU), method="bilinear")
    return jnp.kron(r, r).astype(jnp.bfloat16)


def kernel(x, upper_feat, q_w, q_g, q_b, kv_w, kv_g, kv_b,
           proj_w, proj_g, proj_b, pe_w, pe_g, pe_b, gate_w, gate_b):
    wq = (q_w[:, :, 0, 0] * q_g[:, None]).T
    bq = q_b[None, :]
    wkv = (kv_w[:, :, 0, 0] * kv_g[:, None]).T
    bkv = kv_b[None, :]
    wp = (proj_w[:, :, 0, 0] * proj_g[:, None]).T
    bp = proj_b[None, :]
    peT = pe_w[:, 0].reshape(DIM, 49).T * pe_g[None, :]
    peB = pe_b[None, :]
    gwT = gate_w.T
    gb = gate_b[None, :]
    wkvh = wkv.reshape(DIM, HEADS, 2 * HD).transpose(1, 0, 2)
    bkvh = kv_b.reshape(HEADS, 1, 2 * HD)

    gum = jax.random.gumbel(
        jax.random.key(42), (B, HEADS, NU), F32).transpose(0, 2, 1)

    x_tok = x.reshape(B, DIM, N).transpose(0, 2, 1)
    u_tok = upper_feat.reshape(B, DIM, NU).transpose(0, 2, 1)

    q4, kvu4, gs = _prep_call(x_tok, u_tok, wq, bq, wkv, bkv,
                              _pool_matrix(), gum, _block_mask())
    gs16 = gs.transpose(0, 2, 1).reshape(B * HEADS, NU)
    gx = _sc_call(gs16, x_tok.reshape(B * N, DIM))
    gx4 = gx.reshape(B, HEADS, 4 * TOPK, DIM)
    oa = _attn_call(q4, kvu4, gx4, wkvh, bkvh, gwT, gb)
    out_tok = _epi_call(oa, kvu4, _resize_kron(), wp, bp, peT, peB)
    return out_tok.transpose(0, 2, 1).reshape(B, DIM, H, W)
